# Initial kernel scaffold; baseline (speedup 1.0000x reference)
import jax
import jax.numpy as jnp
from jax.experimental import pallas as pl


def kernel(x, edge_index, W1, b1, W2, b2, Wfc, bfc):
    raise NotImplementedError("write your pallas kernel here")


# speedup vs baseline: 6.9160x; 6.9160x over previous
import functools

import jax
import jax.numpy as jnp
from jax import lax
from jax.experimental import pallas as pl
from jax.experimental.pallas import tpu as pltpu
from jax.experimental.pallas import tpu_sc as plsc

N = 50000
E = 800000
IN_DIM = 3
HID = 128
NCLS = 21

NC, NS = 2, 16
CHUNK = 128
NCH = 6400
EPAD = NCH * CHUNK
NRANGE = 4
RWID = 12544
NPAD = NRANGE * RWID
ACC_ROWS = 12672
TRASH = RWID

BLK = 2000

_MESH = plsc.VectorSubcoreMesh(
    core_axis_name="c", subcore_axis_name="s", num_cores=NC, num_subcores=NS
)


@functools.partial(
    pl.kernel,
    out_type=jax.ShapeDtypeStruct((NC * NPAD,), jnp.float32),
    mesh=_MESH,
    scratch_types=[
        pltpu.VMEM_SHARED((NPAD,), jnp.float32),
        pltpu.VMEM((40, CHUNK), jnp.int32),
        pltpu.VMEM((3136,), jnp.float32),
        pltpu.VMEM((CHUNK,), jnp.float32),
    ],
)
def _deg_kernel(dst_hbm, deg_out, degacc, dstbuf, zbuf, ones):
    c = lax.axis_index("c")
    s = lax.axis_index("s")
    wid = c * NS + s

    @pl.loop(0, 196)
    def _z(i):
        zbuf[pl.ds(i * 16, 16)] = jnp.zeros((16,), jnp.float32)

    @pl.loop(0, 8)
    def _o(i):
        ones[pl.ds(i * 16, 16)] = jnp.ones((16,), jnp.float32)

    pltpu.sync_copy(zbuf.at[pl.ds(0, 3136)], degacc.at[pl.ds(s * 3136, 3136)])
    plsc.subcore_barrier()

    @pl.loop(0, 5)
    def _b(b):
        start = wid * 200 + b * 40
        pltpu.sync_copy(dst_hbm.at[pl.ds(start, 40)], dstbuf)

        @pl.loop(0, 40)
        def _j(j):
            pltpu.sync_copy(ones, degacc.at[dstbuf.at[j]], add=True)

    plsc.subcore_barrier()
    pltpu.sync_copy(degacc.at[pl.ds(s * 3136, 3136)], zbuf)
    pltpu.sync_copy(zbuf, deg_out.at[pl.ds(c * NPAD + s * 3136, 3136)])


@functools.partial(
    pl.kernel,
    out_type=jax.ShapeDtypeStruct((NPAD, HID), jnp.float32),
    mesh=_MESH,
    scratch_types=[
        pltpu.VMEM_SHARED((ACC_ROWS, HID), jnp.float32),
        pltpu.VMEM((40, CHUNK), jnp.int32),
        pltpu.VMEM((40, CHUNK), jnp.int32),
        pltpu.VMEM((CHUNK,), jnp.int32),
        pltpu.VMEM((CHUNK, HID), jnp.float32),
    ],
)
def _agg_kernel(g_hbm, src_hbm, dst_hbm, s_out, acc, srcbuf, dstbuf, lidx, rows):
    c = lax.axis_index("c")
    s = lax.axis_index("s")

    @pl.loop(0, 2)
    def _r(r):
        rbase = (c * 2 + r) * RWID

        @pl.loop(0, 128)
        def _z0(i):
            for k in range(8):
                rows[i, pl.ds(k * 16, 16)] = jnp.zeros((16,), jnp.float32)

        @pl.loop(0, 6)
        def _zz(zi):
            pltpu.sync_copy(rows, acc.at[pl.ds(s * 792 + zi * 128, 128)])

        pltpu.sync_copy(rows.at[pl.ds(0, 24)], acc.at[pl.ds(s * 792 + 768, 24)])
        plsc.subcore_barrier()

        @pl.loop(0, 10)
        def _b(b):
            start = s * 400 + b * 40
            pltpu.sync_copy(src_hbm.at[pl.ds(start, 40)], srcbuf)
            pltpu.sync_copy(dst_hbm.at[pl.ds(start, 40)], dstbuf)

            @pl.loop(0, 40)
            def _j(j):
                for k in range(8):
                    d = dstbuf[j, pl.ds(k * 16, 16)]
                    rel = d - rbase
                    ok = (rel >= 0) & (rel < RWID)
                    t = TRASH + (d & 127)
                    lidx[pl.ds(k * 16, 16)] = jnp.where(ok, rel, t)
                pltpu.sync_copy(g_hbm.at[srcbuf.at[j]], rows)
                pltpu.sync_copy(rows, acc.at[lidx], add=True)

        plsc.subcore_barrier()
        pltpu.sync_copy(
            acc.at[pl.ds(s * 784, 784)],
            s_out.at[pl.ds(rbase + s * 784, 784)],
        )
        plsc.subcore_barrier()


def _tc1_body(x_ref, w1_ref, d0_ref, d1_ref, g_ref, dinv_ref):
    deg = d0_ref[...] + d1_ref[...] + 1.0
    dinv = lax.rsqrt(deg)
    h = jnp.dot(x_ref[...], w1_ref[...], preferred_element_type=jnp.float32)
    g_ref[...] = h * dinv
    dinv_ref[...] = dinv


_tc1 = pl.pallas_call(
    _tc1_body,
    grid=(N // BLK,),
    in_specs=[
        pl.BlockSpec((BLK, IN_DIM), lambda i: (i, 0)),
        pl.BlockSpec((IN_DIM, HID), lambda i: (0, 0)),
        pl.BlockSpec((BLK, 1), lambda i: (i, 0)),
        pl.BlockSpec((BLK, 1), lambda i: (i, 0)),
    ],
    out_specs=[
        pl.BlockSpec((BLK, HID), lambda i: (i, 0)),
        pl.BlockSpec((BLK, 1), lambda i: (i, 0)),
    ],
    out_shape=[
        jax.ShapeDtypeStruct((N, HID), jnp.float32),
        jax.ShapeDtypeStruct((N, 1), jnp.float32),
    ],
)


def _tc2_body(s_ref, g_ref, dinv_ref, b1_ref, w2_ref, g2_ref):
    a = (s_ref[...] + g_ref[...]) * dinv_ref[...] + b1_ref[...]
    h = jnp.maximum(a, 0.0)
    g2_ref[...] = (
        jnp.dot(h, w2_ref[...], preferred_element_type=jnp.float32) * dinv_ref[...]
    )


_tc2 = pl.pallas_call(
    _tc2_body,
    grid=(N // BLK,),
    in_specs=[
        pl.BlockSpec((BLK, HID), lambda i: (i, 0)),
        pl.BlockSpec((BLK, HID), lambda i: (i, 0)),
        pl.BlockSpec((BLK, 1), lambda i: (i, 0)),
        pl.BlockSpec((1, HID), lambda i: (0, 0)),
        pl.BlockSpec((HID, HID), lambda i: (0, 0)),
    ],
    out_specs=pl.BlockSpec((BLK, HID), lambda i: (i, 0)),
    out_shape=jax.ShapeDtypeStruct((N, HID), jnp.float32),
)


def _tc3_body(s_ref, g_ref, dinv_ref, b2_ref, wfc_ref, bfc_ref, out_ref):
    a = (s_ref[...] + g_ref[...]) * dinv_ref[...] + b2_ref[...]
    h = jnp.maximum(a, 0.0)
    out_ref[...] = (
        jnp.dot(h, wfc_ref[...], preferred_element_type=jnp.float32) + bfc_ref[...]
    )


_tc3 = pl.pallas_call(
    _tc3_body,
    grid=(N // BLK,),
    in_specs=[
        pl.BlockSpec((BLK, HID), lambda i: (i, 0)),
        pl.BlockSpec((BLK, HID), lambda i: (i, 0)),
        pl.BlockSpec((BLK, 1), lambda i: (i, 0)),
        pl.BlockSpec((1, HID), lambda i: (0, 0)),
        pl.BlockSpec((HID, NCLS), lambda i: (0, 0)),
        pl.BlockSpec((1, NCLS), lambda i: (0, 0)),
    ],
    out_specs=pl.BlockSpec((BLK, NCLS), lambda i: (i, 0)),
    out_shape=jax.ShapeDtypeStruct((N, NCLS), jnp.float32),
)


def kernel(x, edge_index, W1, b1, W2, b2, Wfc, bfc):
    pad = EPAD - E
    src = edge_index[0]
    dst = edge_index[1]
    pad_src = (jnp.arange(pad, dtype=jnp.int32) * 17) % N
    pad_dst = N + (jnp.arange(pad, dtype=jnp.int32) % (NPAD - N))
    src2 = jnp.concatenate([src, pad_src]).reshape(NCH, CHUNK)
    dst2 = jnp.concatenate([dst, pad_dst]).reshape(NCH, CHUNK)

    deg = _deg_kernel(dst2)
    d0 = deg[0:N, None]
    d1 = deg[NPAD : NPAD + N, None]

    g1, dinv = _tc1(x, W1, d0, d1)
    s1 = _agg_kernel(g1, src2, dst2)
    g2 = _tc2(s1, g1, dinv, b1[None, :], W2)
    s2 = _agg_kernel(g2, src2, dst2)
    logits = _tc3(s2, g2, dinv, b2[None, :], Wfc, bfc[None, :])
    return logits


# speedup vs baseline: 7.0065x; 1.0131x over previous
import functools

import jax
import jax.numpy as jnp
from jax import lax
from jax.experimental import pallas as pl
from jax.experimental.pallas import tpu as pltpu
from jax.experimental.pallas import tpu_sc as plsc

N = 50000
E = 800000
IN_DIM = 3
HID = 128
NCLS = 21

NC, NS = 2, 16
NW = NC * NS
CHUNK = 128
NCH = 6400
EPAD = NCH * CHUNK
NRANGE = 4
RWID = 12544
NPAD = NRANGE * RWID
ACC_ROWS = 12672
TRASH = RWID

BLK = 2000

_MESH = plsc.VectorSubcoreMesh(
    core_axis_name="c", subcore_axis_name="s", num_cores=NC, num_subcores=NS
)


@functools.partial(
    pl.kernel,
    out_type=jax.ShapeDtypeStruct((NC * NPAD,), jnp.float32),
    mesh=_MESH,
    scratch_types=[
        pltpu.VMEM_SHARED((NPAD + 128,), jnp.float32),
        pltpu.VMEM((40, CHUNK), jnp.int32),
        pltpu.VMEM((3136,), jnp.float32),
        pltpu.VMEM((CHUNK,), jnp.float32),
    ],
)
def _deg_kernel(dst_hbm, deg_out, degacc, dstbuf, zbuf, ones):
    c = lax.axis_index("c")
    s = lax.axis_index("s")
    wid = c * NS + s

    @pl.loop(0, 196)
    def _z(i):
        zbuf[pl.ds(i * 16, 16)] = jnp.zeros((16,), jnp.float32)

    @pl.loop(0, 8)
    def _o(i):
        ones[pl.ds(i * 16, 16)] = jnp.ones((16,), jnp.float32)

    pltpu.sync_copy(zbuf.at[pl.ds(0, 3136)], degacc.at[pl.ds(s * 3136, 3136)])

    @pl.when(s == 0)
    def _zx():
        pltpu.sync_copy(zbuf.at[pl.ds(0, 128)], degacc.at[pl.ds(NPAD, 128)])

    plsc.subcore_barrier()

    @pl.loop(0, 5)
    def _b(bi):
        start = wid * 200 + bi * 40
        pltpu.sync_copy(dst_hbm.at[pl.ds(start, 40)], dstbuf)

        @pl.loop(0, 40)
        def _j(j):
            pltpu.sync_copy(ones, degacc.at[dstbuf.at[j]], add=True)

    plsc.subcore_barrier()
    pltpu.sync_copy(degacc.at[pl.ds(s * 3136, 3136)], zbuf)
    pltpu.sync_copy(zbuf, deg_out.at[pl.ds(c * NPAD + s * 3136, 3136)])


@functools.partial(
    pl.kernel,
    out_type=jax.ShapeDtypeStruct((NPAD, HID), jnp.float32),
    mesh=_MESH,
    scratch_types=[
        pltpu.VMEM_SHARED((ACC_ROWS, HID), jnp.float32),
        pltpu.VMEM((40, CHUNK), jnp.int32),
        pltpu.VMEM((40, CHUNK), jnp.int32),
        pltpu.VMEM((CHUNK, HID), jnp.float32),
    ],
)
def _agg_kernel(g_hbm, src_hbm, dst_hbm, s_out, acc, srcbuf, dstbuf, rows):
    c = lax.axis_index("c")
    s = lax.axis_index("s")
    iota16 = lax.iota(jnp.int32, 16)

    @pl.loop(0, 2)
    def _r(r):
        rng = c * 2 + r
        base = rng * RWID

        @pl.loop(0, 128)
        def _z0(i):
            for k in range(8):
                rows[i, pl.ds(k * 16, 16)] = jnp.zeros((16,), jnp.float32)

        @pl.loop(0, 6)
        def _zz(zi):
            pltpu.sync_copy(rows, acc.at[pl.ds(s * 792 + zi * 128, 128)])

        pltpu.sync_copy(rows.at[pl.ds(0, 24)], acc.at[pl.ds(s * 792 + 768, 24)])
        plsc.subcore_barrier()

        @pl.loop(0, 10)
        def _b(i):
            start = s * 400 + i * 40
            pltpu.sync_copy(src_hbm.at[pl.ds(start, 40)], srcbuf)
            pltpu.sync_copy(dst_hbm.at[pl.ds(start, 40)], dstbuf)

            @pl.loop(0, 40)
            def _j(j):
                for k in range(8):
                    d = dstbuf[j, pl.ds(k * 16, 16)]
                    m = (d >= base) & (d < base + RWID)
                    trash = TRASH + ((iota16 * 5 + (s * 8 + k) * 7) & 127)
                    dstbuf[j, pl.ds(k * 16, 16)] = jnp.where(m, d - base, trash)

                pltpu.sync_copy(g_hbm.at[srcbuf.at[j]], rows)
                pltpu.sync_copy(rows, acc.at[dstbuf.at[j]], add=True)

        plsc.subcore_barrier()
        pltpu.sync_copy(
            acc.at[pl.ds(s * 784, 784)],
            s_out.at[pl.ds(base + s * 784, 784)],
        )
        plsc.subcore_barrier()


def _tc1_body(x_ref, w1_ref, d0_ref, d1_ref, g_ref, dinv_ref):
    deg = d0_ref[...] + d1_ref[...] + 1.0
    dinv = lax.rsqrt(deg)
    h = jnp.dot(x_ref[...], w1_ref[...], preferred_element_type=jnp.float32)
    g_ref[...] = h * dinv
    dinv_ref[...] = dinv


_tc1 = pl.pallas_call(
    _tc1_body,
    grid=(N // BLK,),
    in_specs=[
        pl.BlockSpec((BLK, IN_DIM), lambda i: (i, 0)),
        pl.BlockSpec((IN_DIM, HID), lambda i: (0, 0)),
        pl.BlockSpec((BLK, 1), lambda i: (i, 0)),
        pl.BlockSpec((BLK, 1), lambda i: (i, 0)),
    ],
    out_specs=[
        pl.BlockSpec((BLK, HID), lambda i: (i, 0)),
        pl.BlockSpec((BLK, 1), lambda i: (i, 0)),
    ],
    out_shape=[
        jax.ShapeDtypeStruct((N, HID), jnp.float32),
        jax.ShapeDtypeStruct((N, 1), jnp.float32),
    ],
)


def _tc2_body(s_ref, g_ref, dinv_ref, b1_ref, w2_ref, g2_ref):
    a = (s_ref[...] + g_ref[...]) * dinv_ref[...] + b1_ref[...]
    h = jnp.maximum(a, 0.0)
    g2_ref[...] = (
        jnp.dot(h, w2_ref[...], preferred_element_type=jnp.float32) * dinv_ref[...]
    )


_tc2 = pl.pallas_call(
    _tc2_body,
    grid=(N // BLK,),
    in_specs=[
        pl.BlockSpec((BLK, HID), lambda i: (i, 0)),
        pl.BlockSpec((BLK, HID), lambda i: (i, 0)),
        pl.BlockSpec((BLK, 1), lambda i: (i, 0)),
        pl.BlockSpec((1, HID), lambda i: (0, 0)),
        pl.BlockSpec((HID, HID), lambda i: (0, 0)),
    ],
    out_specs=pl.BlockSpec((BLK, HID), lambda i: (i, 0)),
    out_shape=jax.ShapeDtypeStruct((N, HID), jnp.float32),
)


def _tc3_body(s_ref, g_ref, dinv_ref, b2_ref, wfc_ref, bfc_ref, out_ref):
    a = (s_ref[...] + g_ref[...]) * dinv_ref[...] + b2_ref[...]
    h = jnp.maximum(a, 0.0)
    out_ref[...] = (
        jnp.dot(h, wfc_ref[...], preferred_element_type=jnp.float32) + bfc_ref[...]
    )


_tc3 = pl.pallas_call(
    _tc3_body,
    grid=(N // BLK,),
    in_specs=[
        pl.BlockSpec((BLK, HID), lambda i: (i, 0)),
        pl.BlockSpec((BLK, HID), lambda i: (i, 0)),
        pl.BlockSpec((BLK, 1), lambda i: (i, 0)),
        pl.BlockSpec((1, HID), lambda i: (0, 0)),
        pl.BlockSpec((HID, NCLS), lambda i: (0, 0)),
        pl.BlockSpec((1, NCLS), lambda i: (0, 0)),
    ],
    out_specs=pl.BlockSpec((BLK, NCLS), lambda i: (i, 0)),
    out_shape=jax.ShapeDtypeStruct((N, NCLS), jnp.float32),
)


def kernel(x, edge_index, W1, b1, W2, b2, Wfc, bfc):
    pad = EPAD - E
    src = edge_index[0]
    dst = edge_index[1]
    pad_src = (jnp.arange(pad, dtype=jnp.int32) * 17) % N
    pad_dst = NPAD + (jnp.arange(pad, dtype=jnp.int32) % 128)
    src2 = jnp.concatenate([src, pad_src]).reshape(NCH, CHUNK)
    dst2 = jnp.concatenate([dst, pad_dst]).reshape(NCH, CHUNK)

    deg = _deg_kernel(dst2)
    d0 = deg[0:N, None]
    d1 = deg[NPAD : NPAD + N, None]

    g1, dinv = _tc1(x, W1, d0, d1)
    s1 = _agg_kernel(g1, src2, dst2)
    g2 = _tc2(s1, g1, dinv, b1[None, :], W2)
    s2 = _agg_kernel(g2, src2, dst2)
    logits = _tc3(s2, g2, dinv, b2[None, :], Wfc, bfc[None, :])
    return logits
